# Initial kernel scaffold; baseline (speedup 1.0000x reference)
import jax
import jax.numpy as jnp
from jax.experimental import pallas as pl


def kernel(x, record_len, query, context, Wq, bq, Wc, bc):
    raise NotImplementedError("write your pallas kernel here")


# speedup vs baseline: 2.0734x; 2.0734x over previous
import functools

import jax
import jax.numpy as jnp
from jax import lax
from jax.experimental import pallas as pl
from jax.experimental.pallas import tpu as pltpu
from jax.experimental.pallas import tpu_sc as plsc

TOTAL = 16384
NSEG = 16
D = 1024
CSC = 512
RPW = 1024
CH = 128
NCH = RPW // CH
NG = CSC // 16
NEG = float("-inf")


def _body(x_hbm, starts_hbm, ends_hbm, out_hbm,
          bnds_s, buf, acc, mbuf, macc, spmem, sem):
    cid = lax.axis_index("c")
    sid = lax.axis_index("s")
    col0 = cid * CSC
    row0 = sid * RPW

    pltpu.sync_copy(starts_hbm, bnds_s.at[0])
    pltpu.sync_copy(ends_hbm, bnds_s.at[1])
    starts_v = bnds_s[0]
    ends_v = bnds_s[1]

    neg = jnp.full((16,), NEG, jnp.float32)
    for s in range(NSEG):
        for g in range(NG):
            acc[s, pl.ds(g * 16, 16)] = neg

    def chunk_body(ch, carry):
        r0 = row0 + ch * CH
        pltpu.async_copy(
            x_hbm.at[pl.ds(r0, CH), pl.ds(col0, CSC)], buf, sem
        ).wait()
        for s in range(NSEG):
            lo = jnp.clip(starts_v[s] - r0, 0, CH)
            hi = jnp.clip(ends_v[s] - r0, 0, CH)

            @pl.when(hi > lo)
            def _process(s=s, lo=lo, hi=hi):
                accv = tuple(acc[s, pl.ds(g * 16, 16)] for g in range(NG))

                def row_body(r, c):
                    return tuple(
                        jnp.maximum(c[g], buf[r, pl.ds(g * 16, 16)])
                        for g in range(NG)
                    )

                accv = lax.fori_loop(lo, hi, row_body, accv)
                for g in range(NG):
                    acc[s, pl.ds(g * 16, 16)] = accv[g]
        return carry

    lax.fori_loop(0, NCH, chunk_body, 0)

    pltpu.sync_copy(acc, spmem.at[sid])
    plsc.subcore_barrier()

    @pl.when(sid < 4)
    def _merge():
        pltpu.sync_copy(spmem.at[0, :, pl.ds(sid * 128, 128)], macc)

        def merge_body(t, carry):
            pltpu.sync_copy(spmem.at[t, :, pl.ds(sid * 128, 128)], mbuf)
            for r in range(NSEG):
                for g in range(8):
                    macc[r, pl.ds(g * 16, 16)] = jnp.maximum(
                        macc[r, pl.ds(g * 16, 16)],
                        mbuf[r, pl.ds(g * 16, 16)],
                    )
            return carry

        lax.fori_loop(1, 16, merge_body, 0)
        pltpu.sync_copy(macc, out_hbm.at[:, pl.ds(col0 + sid * 128, 128)])


@jax.jit
def _seg_max(x, starts, ends):
    mesh = plsc.VectorSubcoreMesh(core_axis_name="c", subcore_axis_name="s")
    return pl.kernel(
        _body,
        out_type=jax.ShapeDtypeStruct((NSEG, D), jnp.float32),
        mesh=mesh,
        scratch_types=[
            pltpu.VMEM((2, NSEG), jnp.int32),
            pltpu.VMEM((CH, CSC), jnp.float32),
            pltpu.VMEM((NSEG, CSC), jnp.float32),
            pltpu.VMEM((NSEG, 128), jnp.float32),
            pltpu.VMEM((NSEG, 128), jnp.float32),
            pltpu.VMEM_SHARED((16, NSEG, CSC), jnp.float32),
            pltpu.SemaphoreType.DMA,
        ],
    )(x, starts, ends)


def kernel(x, record_len, query, context, Wq, bq, Wc, bc):
    cum = jnp.cumsum(record_len.astype(jnp.int32))
    split = jnp.clip(cum[: NSEG - 1], 0, TOTAL)
    starts = jnp.concatenate([jnp.zeros((1,), jnp.int32), split])
    ends = jnp.concatenate([split, jnp.full((1,), TOTAL, jnp.int32)])
    return _seg_max(x, starts, ends)


# speedup vs baseline: 2.1348x; 1.0296x over previous
import functools

import jax
import jax.numpy as jnp
from jax import lax
from jax.experimental import pallas as pl
from jax.experimental.pallas import tpu as pltpu
from jax.experimental.pallas import tpu_sc as plsc

TOTAL = 16384
NSEG = 16
D = 1024
CSC = 512
RPW = 1024
CH = 64
NCH = RPW // CH
NG = CSC // 16
NEG = float("-inf")


def _reduce_chunk(buf, acc, r0, starts_v, ends_v):
    for s in range(NSEG):
        lo = jnp.clip(starts_v[s] - r0, 0, CH)
        hi = jnp.clip(ends_v[s] - r0, 0, CH)

        @pl.when(hi > lo)
        def _process(s=s, lo=lo, hi=hi):
            accv = tuple(acc[s, pl.ds(g * 16, 16)] for g in range(NG))

            def row_body(r, c):
                return tuple(
                    jnp.maximum(c[g], buf[r, pl.ds(g * 16, 16)])
                    for g in range(NG)
                )

            accv = lax.fori_loop(lo, hi, row_body, accv)
            for g in range(NG):
                acc[s, pl.ds(g * 16, 16)] = accv[g]


def _body(x_hbm, starts_hbm, ends_hbm, out_hbm,
          bnds_s, buf0, buf1, acc, mbuf, macc, spmem, sem0, sem1):
    cid = lax.axis_index("c")
    sid = lax.axis_index("s")
    col0 = cid * CSC
    row0 = sid * RPW

    def chunk_src(i):
        return x_hbm.at[pl.ds(row0 + i * CH, CH), pl.ds(col0, CSC)]

    pltpu.sync_copy(starts_hbm, bnds_s.at[0])
    pltpu.sync_copy(ends_hbm, bnds_s.at[1])
    starts_v = bnds_s[0]
    ends_v = bnds_s[1]

    pltpu.async_copy(chunk_src(0), buf0, sem0)

    neg = jnp.full((16,), NEG, jnp.float32)
    for s in range(NSEG):
        for g in range(NG):
            acc[s, pl.ds(g * 16, 16)] = neg

    def pair_body(g, carry):
        e = 2 * g
        pltpu.async_copy(chunk_src(e + 1), buf1, sem1)
        pltpu.make_async_copy(chunk_src(0), buf0, sem0).wait()
        _reduce_chunk(buf0, acc, row0 + e * CH, starts_v, ends_v)

        @pl.when(e + 2 < NCH)
        def _prefetch():
            pltpu.async_copy(chunk_src(e + 2), buf0, sem0)

        pltpu.make_async_copy(chunk_src(0), buf1, sem1).wait()
        _reduce_chunk(buf1, acc, row0 + (e + 1) * CH, starts_v, ends_v)
        return carry

    lax.fori_loop(0, NCH // 2, pair_body, 0)

    pltpu.sync_copy(acc, spmem.at[sid])
    plsc.subcore_barrier()

    @pl.when(sid < 4)
    def _merge():
        pltpu.sync_copy(spmem.at[0, :, pl.ds(sid * 128, 128)], macc)

        def merge_body(t, carry):
            pltpu.sync_copy(spmem.at[t, :, pl.ds(sid * 128, 128)], mbuf)
            for r in range(NSEG):
                for g in range(8):
                    macc[r, pl.ds(g * 16, 16)] = jnp.maximum(
                        macc[r, pl.ds(g * 16, 16)],
                        mbuf[r, pl.ds(g * 16, 16)],
                    )
            return carry

        lax.fori_loop(1, 16, merge_body, 0)
        pltpu.sync_copy(macc, out_hbm.at[:, pl.ds(col0 + sid * 128, 128)])


@jax.jit
def _seg_max(x, starts, ends):
    mesh = plsc.VectorSubcoreMesh(core_axis_name="c", subcore_axis_name="s")
    return pl.kernel(
        _body,
        out_type=jax.ShapeDtypeStruct((NSEG, D), jnp.float32),
        mesh=mesh,
        scratch_types=[
            pltpu.VMEM((2, NSEG), jnp.int32),
            pltpu.VMEM((CH, CSC), jnp.float32),
            pltpu.VMEM((CH, CSC), jnp.float32),
            pltpu.VMEM((NSEG, CSC), jnp.float32),
            pltpu.VMEM((NSEG, 128), jnp.float32),
            pltpu.VMEM((NSEG, 128), jnp.float32),
            pltpu.VMEM_SHARED((16, NSEG, CSC), jnp.float32),
            pltpu.SemaphoreType.DMA,
            pltpu.SemaphoreType.DMA,
        ],
    )(x, starts, ends)


def kernel(x, record_len, query, context, Wq, bq, Wc, bc):
    cum = jnp.cumsum(record_len.astype(jnp.int32))
    split = jnp.clip(cum[: NSEG - 1], 0, TOTAL)
    starts = jnp.concatenate([jnp.zeros((1,), jnp.int32), split])
    ends = jnp.concatenate([split, jnp.full((1,), TOTAL, jnp.int32)])
    return _seg_max(x, starts, ends)


# speedup vs baseline: 2.1915x; 1.0266x over previous
import functools

import jax
import jax.numpy as jnp
from jax import lax
from jax.experimental import pallas as pl
from jax.experimental.pallas import tpu as pltpu
from jax.experimental.pallas import tpu_sc as plsc

TOTAL = 16384
NSEG = 16
D = 1024
CSC = 512
RPW = 1024
CH = 64
NCH = RPW // CH
NG = CSC // 16
NEG = float("-inf")


def _reduce_chunk(buf, acc, r0, starts_v, ends_v):
    for s in range(NSEG):
        lo = jnp.clip(starts_v[s] - r0, 0, CH)
        hi = jnp.clip(ends_v[s] - r0, 0, CH)

        @pl.when(hi > lo)
        def _process(s=s, lo=lo, hi=hi):
            for h in range(2):
                hg = NG // 2
                c0 = h * hg * 16
                accv = tuple(
                    acc[s, pl.ds(c0 + g * 16, 16)] for g in range(hg)
                )

                def row_body(r, c, c0=c0, hg=hg):
                    return tuple(
                        jnp.maximum(c[g], buf[r, pl.ds(c0 + g * 16, 16)])
                        for g in range(hg)
                    )

                accv = lax.fori_loop(lo, hi, row_body, accv)
                for g in range(hg):
                    acc[s, pl.ds(c0 + g * 16, 16)] = accv[g]


def _body(x_hbm, starts_hbm, ends_hbm, out_hbm,
          bnds_s, buf0, buf1, acc, mbuf, macc, spmem, sem0, sem1):
    cid = lax.axis_index("c")
    sid = lax.axis_index("s")
    col0 = cid * CSC
    row0 = sid * RPW

    def chunk_src(i):
        return x_hbm.at[pl.ds(row0 + i * CH, CH), pl.ds(col0, CSC)]

    pltpu.sync_copy(starts_hbm, bnds_s.at[0])
    pltpu.sync_copy(ends_hbm, bnds_s.at[1])
    starts_v = bnds_s[0]
    ends_v = bnds_s[1]

    pltpu.async_copy(chunk_src(0), buf0, sem0)

    neg = jnp.full((16,), NEG, jnp.float32)
    for s in range(NSEG):
        for g in range(NG):
            acc[s, pl.ds(g * 16, 16)] = neg

    def pair_body(g, carry):
        e = 2 * g
        pltpu.async_copy(chunk_src(e + 1), buf1, sem1)
        pltpu.make_async_copy(chunk_src(0), buf0, sem0).wait()
        _reduce_chunk(buf0, acc, row0 + e * CH, starts_v, ends_v)

        @pl.when(e + 2 < NCH)
        def _prefetch():
            pltpu.async_copy(chunk_src(e + 2), buf0, sem0)

        pltpu.make_async_copy(chunk_src(0), buf1, sem1).wait()
        _reduce_chunk(buf1, acc, row0 + (e + 1) * CH, starts_v, ends_v)
        return carry

    lax.fori_loop(0, NCH // 2, pair_body, 0)

    pltpu.sync_copy(acc, spmem.at[sid])
    plsc.subcore_barrier()

    @pl.when(sid < 4)
    def _merge():
        pltpu.sync_copy(spmem.at[0, :, pl.ds(sid * 128, 128)], macc)

        def merge_body(t, carry):
            pltpu.sync_copy(spmem.at[t, :, pl.ds(sid * 128, 128)], mbuf)
            for r in range(NSEG):
                for g in range(8):
                    macc[r, pl.ds(g * 16, 16)] = jnp.maximum(
                        macc[r, pl.ds(g * 16, 16)],
                        mbuf[r, pl.ds(g * 16, 16)],
                    )
            return carry

        lax.fori_loop(1, 16, merge_body, 0)
        pltpu.sync_copy(macc, out_hbm.at[:, pl.ds(col0 + sid * 128, 128)])


@jax.jit
def _seg_max(x, starts, ends):
    mesh = plsc.VectorSubcoreMesh(core_axis_name="c", subcore_axis_name="s")
    return pl.kernel(
        _body,
        out_type=jax.ShapeDtypeStruct((NSEG, D), jnp.float32),
        mesh=mesh,
        scratch_types=[
            pltpu.VMEM((2, NSEG), jnp.int32),
            pltpu.VMEM((CH, CSC), jnp.float32),
            pltpu.VMEM((CH, CSC), jnp.float32),
            pltpu.VMEM((NSEG, CSC), jnp.float32),
            pltpu.VMEM((NSEG, 128), jnp.float32),
            pltpu.VMEM((NSEG, 128), jnp.float32),
            pltpu.VMEM_SHARED((16, NSEG, CSC), jnp.float32),
            pltpu.SemaphoreType.DMA,
            pltpu.SemaphoreType.DMA,
        ],
    )(x, starts, ends)


def kernel(x, record_len, query, context, Wq, bq, Wc, bc):
    cum = jnp.cumsum(record_len.astype(jnp.int32))
    split = jnp.clip(cum[: NSEG - 1], 0, TOTAL)
    starts = jnp.concatenate([jnp.zeros((1,), jnp.int32), split])
    ends = jnp.concatenate([split, jnp.full((1,), TOTAL, jnp.int32)])
    return _seg_max(x, starts, ends)
